# baseline (speedup 1.0000x reference)
import functools

import jax
import jax.numpy as jnp
from jax import lax
from jax.experimental import pallas as pl
from jax.experimental.pallas import tpu as pltpu
from jax.experimental.pallas import tpu_sc as plsc

_D = 17
_NPAIR = _D * _D * _D * (_D - 1)
_L = 16
_NW = 32
_P = 8192


def _interp_body(x_hbm, lut_hbm, out_hbm, lut_v, x0_v, x1_v, x2_v, x3_v, o_v):
    nb = x_hbm.shape[0]
    hw = x_hbm.shape[2]
    wpb = _NW // nb
    span = hw // wpb

    wid = lax.axis_index("s") * 2 + lax.axis_index("c")
    batch = wid // wpb
    chunk = wid % wpb

    pltpu.sync_copy(lut_hbm.at[batch], lut_v)

    xbufs = (x0_v, x1_v, x2_v, x3_v)

    def prep(v):
        xs = jnp.minimum(jnp.maximum(v, 0.0), 1.0) * jnp.float32(_D - 1)
        i0 = jnp.minimum(xs.astype(jnp.int32), _D - 2)
        return i0, xs - i0.astype(jnp.float32)

    for t in range(span // _P):
        off = batch * hw + chunk * span + t * _P
        for ch in range(4):
            pltpu.sync_copy(x_hbm.at[batch, ch, pl.ds(chunk * span + t * _P, _P)],
                            xbufs[ch])

        @plsc.parallel_loop(0, _P // _L, 1, unroll=4)
        def step(i):
            sl = pl.ds(i * _L, _L)
            ia, fa = prep(x0_v[sl])
            ib, fb = prep(x1_v[sl])
            ic, fc = prep(x2_v[sl])
            idd, fd = prep(x3_v[sl])
            lin = (((ia * _D + ib) * _D + ic) * (_D - 1)) + idd
            vdd = []
            for da in (0, 1):
                for db in (0, 1):
                    for dc in (0, 1):
                        base = lin + (da * (_D * _D * (_D - 1))
                                      + db * (_D * (_D - 1)) + dc * (_D - 1))
                        g = plsc.load_gather(lut_v, [base])
                        gb = plsc.bitcast(g, jnp.bfloat16)
                        v0, v1 = plsc.unpack(
                            gb, format=plsc.PackFormat.INTERLEAVED,
                            preferred_element_type=jnp.float32)
                        vdd.append(v0 + fd * (v1 - v0))
            vc = [vdd[k] + fc * (vdd[k + 1] - vdd[k]) for k in (0, 2, 4, 6)]
            vb = [vc[0] + fb * (vc[1] - vc[0]), vc[2] + fb * (vc[3] - vc[2])]
            o_v[sl] = vb[0] + fa * (vb[1] - vb[0])

        pltpu.sync_copy(o_v, out_hbm.at[pl.ds(off, _P)])


def kernel(x, LUT):
    nb, nc, h, w = x.shape
    hw = h * w
    lut4 = LUT[:, 0]
    lo = jax.lax.bitcast_convert_type(
        lut4[..., :-1].astype(jnp.bfloat16), jnp.uint16).astype(jnp.uint32)
    hi = jax.lax.bitcast_convert_type(
        lut4[..., 1:].astype(jnp.bfloat16), jnp.uint16).astype(jnp.uint32)
    lut_pairs = jax.lax.bitcast_convert_type(
        lo | (hi << 16), jnp.int32).reshape(nb, _NPAIR)
    x_flat = x.reshape(nb, nc, hw)

    run = functools.partial(
        pl.kernel,
        out_type=jax.ShapeDtypeStruct((nb * hw,), jnp.float32),
        mesh=plsc.VectorSubcoreMesh(core_axis_name="c", subcore_axis_name="s"),
        compiler_params=pltpu.CompilerParams(needs_layout_passes=False),
        scratch_types=[
            pltpu.VMEM((_NPAIR,), jnp.int32),
            pltpu.VMEM((_P,), jnp.float32),
            pltpu.VMEM((_P,), jnp.float32),
            pltpu.VMEM((_P,), jnp.float32),
            pltpu.VMEM((_P,), jnp.float32),
            pltpu.VMEM((_P,), jnp.float32),
        ],
    )(_interp_body)
    out = run(x_flat, lut_pairs)
    return out.reshape(nb, 1, h, w)


# speedup vs baseline: 1.2533x; 1.2533x over previous
import functools

import jax
import jax.numpy as jnp
from jax import lax
from jax.experimental import pallas as pl
from jax.experimental.pallas import tpu as pltpu
from jax.experimental.pallas import tpu_sc as plsc

_D = 17
_NLUT = _D ** 4
_LUTP = _NLUT + 7
_L = 16
_NW = 32
_ROWS = 16
_W = 512
_P = _ROWS * _W


def _interp_body(x_hbm, lut_hbm, out_hbm, lut_v, x0_v, x1_v, x2_v, x3_v, o_v):
    nb = x_hbm.shape[0]
    h = x_hbm.shape[2]
    wpb = _NW // nb
    rows_per_w = h // wpb

    wid = lax.axis_index("s") * 2 + lax.axis_index("c")
    batch = wid // wpb
    chunk = wid % wpb

    pltpu.sync_copy(lut_hbm.at[batch], lut_v)

    xbufs = (x0_v, x1_v, x2_v, x3_v)

    def prep(v):
        xs = jnp.minimum(jnp.maximum(v, 0.0) * jnp.float32(_D - 1),
                         jnp.float32(_D - 1))
        i0 = jnp.minimum(xs.astype(jnp.int32), _D - 2)
        return i0, xs - i0.astype(jnp.float32)

    for t in range(rows_per_w // _ROWS):
        r0 = chunk * rows_per_w + t * _ROWS
        for ch in range(4):
            pltpu.sync_copy(x_hbm.at[batch, ch, pl.ds(r0, _ROWS)], xbufs[ch])

        @plsc.parallel_loop(0, _P // _L, 1, unroll=4)
        def step(i):
            row = lax.shift_right_logical(i, 5)
            col = lax.shift_left(jnp.bitwise_and(i, 31), 4)
            sl = pl.ds(col, _L)
            ia, fa = prep(x0_v[row, sl])
            ib, fb = prep(x1_v[row, sl])
            ic, fc = prep(x2_v[row, sl])
            idd, fd = prep(x3_v[row, sl])
            lin = ((ia * _D + ib) * _D + ic) * _D + idd
            vdd = []
            for da in (0, 1):
                for db in (0, 1):
                    for dc in (0, 1):
                        base = lin + (da * _D ** 3 + db * _D ** 2 + dc * _D)
                        g = plsc.load_gather(lut_v, [base])
                        gb = plsc.bitcast(g, jnp.bfloat16)
                        v0, v1 = plsc.unpack(
                            gb, format=plsc.PackFormat.INTERLEAVED,
                            preferred_element_type=jnp.float32)
                        vdd.append(v0 + fd * (v1 - v0))
            vc = [vdd[k] + fc * (vdd[k + 1] - vdd[k]) for k in (0, 2, 4, 6)]
            vb = [vc[0] + fb * (vc[1] - vc[0]), vc[2] + fb * (vc[3] - vc[2])]
            o_v[row, sl] = vb[0] + fa * (vb[1] - vb[0])

        pltpu.sync_copy(o_v, out_hbm.at[batch, 0, pl.ds(r0, _ROWS)])


def kernel(x, LUT):
    nb, nc, h, w = x.shape

    bits = jax.lax.bitcast_convert_type(
        LUT.reshape(nb, _NLUT).astype(jnp.bfloat16), jnp.uint16)
    lo = bits.astype(jnp.uint32)
    hi = jnp.pad(bits[:, 1:], ((0, 0), (0, 1))).astype(jnp.uint32)
    lut_pairs = jnp.pad(
        jax.lax.bitcast_convert_type(lo | (hi << 16), jnp.int32),
        ((0, 0), (0, _LUTP - _NLUT)))

    run = functools.partial(
        pl.kernel,
        out_type=jax.ShapeDtypeStruct((nb, 1, h, w), jnp.float32),
        mesh=plsc.VectorSubcoreMesh(core_axis_name="c", subcore_axis_name="s"),
        compiler_params=pltpu.CompilerParams(needs_layout_passes=False),
        scratch_types=[
            pltpu.VMEM((_LUTP,), jnp.int32),
            pltpu.VMEM((_ROWS, _W), jnp.float32),
            pltpu.VMEM((_ROWS, _W), jnp.float32),
            pltpu.VMEM((_ROWS, _W), jnp.float32),
            pltpu.VMEM((_ROWS, _W), jnp.float32),
            pltpu.VMEM((_ROWS, _W), jnp.float32),
        ],
    )(_interp_body)
    return run(x, lut_pairs)
